# speedup vs baseline: 2.7267x; 2.7267x over previous
import functools

import jax
import jax.numpy as jnp
from jax import lax
from jax.experimental import pallas as pl
from jax.experimental.pallas import tpu as pltpu
from jax.experimental.pallas import tpu_sc as plsc

N = 10000
M = 32
AF = 128
NF = 16
H = 128
NCRY = 100
APC = 100

NCORE = 2
NSUB = 16
NWORK = NCORE * NSUB
LANES = 16

APW = 320
APAD = NWORK * APW
CA = 4
ECH = CA * M
NCHUNK = APW // CA

GPW = 4
GPAD = NWORK * GPW
APC_PAD = 104

C0 = 0.28209479177387814
EDGE_SCALE = C0 / (float(AF) ** 0.5) / float(M)

_FCH = AF // LANES


def _conv_gather(x, idx_pad, w_pad):
    mesh = plsc.VectorSubcoreMesh(core_axis_name="c", subcore_axis_name="s")

    @functools.partial(
        pl.kernel,
        out_type=jax.ShapeDtypeStruct((APAD, AF), jnp.float32),
        mesh=mesh,
        scratch_types=[
            pltpu.VMEM((ECH,), jnp.int32),
            pltpu.VMEM((ECH,), jnp.float32),
            pltpu.VMEM((ECH, AF), jnp.float32),
            pltpu.VMEM((CA, AF), jnp.float32),
            pltpu.SemaphoreType.DMA,
        ],
    )
    def k(x_hbm, idx_hbm, w_hbm, out_hbm, idx_v, w_v, rows_v, out_v, sem):
        wid = lax.axis_index("s") * NCORE + lax.axis_index("c")
        ebase = wid * (APW * M)
        abase = wid * APW

        def chunk(c, _):
            eoff = ebase + c * ECH
            pltpu.sync_copy(idx_hbm.at[pl.ds(eoff, ECH)], idx_v)
            pltpu.sync_copy(w_hbm.at[pl.ds(eoff, ECH)], w_v)
            pltpu.async_copy(x_hbm.at[idx_v], rows_v, sem).wait()

            def atom(a, _):
                e0 = a * M
                wv0 = w_v[pl.ds(e0, LANES)]
                wv1 = w_v[pl.ds(e0 + LANES, LANES)]
                accs = [jnp.zeros((LANES,), jnp.float32) for _ in range(_FCH)]
                for e in range(M):
                    wv = wv0 if e < LANES else wv1
                    lane = jnp.full((LANES,), e % LANES, jnp.int32)
                    wb = wv.at[lane].get(mode="promise_in_bounds")
                    for f in range(_FCH):
                        row = rows_v[e0 + e, pl.ds(f * LANES, LANES)]
                        accs[f] = accs[f] + wb * row
                for f in range(_FCH):
                    out_v[a, pl.ds(f * LANES, LANES)] = accs[f]
                return 0

            lax.fori_loop(0, CA, atom, 0)
            pltpu.sync_copy(out_v, out_hbm.at[pl.ds(abase + c * CA, CA)])
            return 0

        lax.fori_loop(0, NCHUNK, chunk, 0)

    return k(x, idx_pad, w_pad)


def _crystal_pool(x, cidx_pad):
    mesh = plsc.VectorSubcoreMesh(core_axis_name="c", subcore_axis_name="s")
    cidx_flat = cidx_pad.reshape(-1)

    @functools.partial(
        pl.kernel,
        out_type=jax.ShapeDtypeStruct((GPAD, AF), jnp.float32),
        mesh=mesh,
        scratch_types=[
            pltpu.VMEM((APC_PAD,), jnp.int32),
            pltpu.VMEM((APC_PAD, AF), jnp.float32),
            pltpu.VMEM((GPW, AF), jnp.float32),
            pltpu.SemaphoreType.DMA,
        ],
    )
    def k(x_hbm, cidx_hbm, out_hbm, idx_v, rows_v, out_v, sem):
        wid = lax.axis_index("s") * NCORE + lax.axis_index("c")
        gbase = wid * GPW

        def group(g, _):
            pltpu.sync_copy(cidx_hbm.at[pl.ds((gbase + g) * APC_PAD, APC_PAD)], idx_v)
            pltpu.async_copy(x_hbm.at[idx_v], rows_v, sem).wait()
            scale = jnp.float32(1.0 / APC)
            for f in range(_FCH):
                acc = jnp.zeros((LANES,), jnp.float32)
                for e in range(APC):
                    acc = acc + rows_v[e, pl.ds(f * LANES, LANES)]
                out_v[g, pl.ds(f * LANES, LANES)] = acc * scale
            return 0

        lax.fori_loop(0, GPW, group, 0)
        pltpu.sync_copy(out_v, out_hbm.at[pl.ds(gbase, GPW)])

    return k(x, cidx_flat)


def _emb_body(af_ref, w_ref, b_ref, o_ref):
    o_ref[...] = (
        jnp.dot(af_ref[...], w_ref[...], preferred_element_type=jnp.float32)
        + b_ref[...]
    )


def _edge_w_body(nbr_ref, w1_ref, b1_ref, w2_ref, o_ref):
    nbr = nbr_ref[...]
    outs = []
    for l in range(3):
        h = jax.nn.softplus(
            jnp.dot(nbr, w1_ref[l], preferred_element_type=jnp.float32)
            + b1_ref[l]
        )
        s = jnp.dot(h, w2_ref[l], preferred_element_type=jnp.float32)
        outs.append(s.T)
    o_ref[...] = jnp.concatenate(outs, axis=0)


def _bn_body(x_ref, g_ref, wtp_ref, gm_ref, bt_ref, o_ref):
    pre = x_ref[...] + jnp.dot(
        g_ref[...], wtp_ref[...], preferred_element_type=jnp.float32
    )
    mean = jnp.mean(pre, axis=0, keepdims=True)
    d = pre - mean
    var = jnp.mean(d * d, axis=0, keepdims=True)
    o_ref[...] = jax.nn.softplus(
        d * lax.rsqrt(var + 1e-5) * gm_ref[...] + bt_ref[...]
    )


def _readout_body(c_ref, wfc_ref, bfc_ref, wout_ref, bout_ref, o_ref, h_ref):
    h = jax.nn.softplus(
        jnp.dot(c_ref[...], wfc_ref[...], preferred_element_type=jnp.float32)
        + bfc_ref[...]
    )
    h_ref[...] = h
    o_ref[...] = (
        jnp.dot(h, wout_ref[...], preferred_element_type=jnp.float32)
        + bout_ref[...]
    )


def kernel(atom_fea, nbr_fea, nbr_idx, crystal_atom_idx, pos, params):
    del pos

    x = pl.pallas_call(
        _emb_body,
        out_shape=jax.ShapeDtypeStruct((N, AF), jnp.float32),
    )(atom_fea, params["W_emb"], params["b_emb"].reshape(1, AF))

    w1s = jnp.stack([p["W1"] for p in params["convs"]])
    b1s = jnp.stack([p["b1"].reshape(1, NF) for p in params["convs"]])
    w2s = jnp.stack(
        [p["W2"][:, :1] * EDGE_SCALE for p in params["convs"]]
    )
    b2s = jnp.stack([p["b2"][0] * EDGE_SCALE for p in params["convs"]])

    E0 = N * M
    EB = 32000
    w_all = pl.pallas_call(
        _edge_w_body,
        grid=(E0 // EB,),
        in_specs=[
            pl.BlockSpec((EB, NF), lambda i: (i, 0)),
            pl.BlockSpec((3, NF, NF), lambda i: (0, 0, 0)),
            pl.BlockSpec((3, 1, NF), lambda i: (0, 0, 0)),
            pl.BlockSpec((3, NF, 1), lambda i: (0, 0, 0)),
        ],
        out_specs=pl.BlockSpec((3, EB), lambda i: (0, i)),
        out_shape=jax.ShapeDtypeStruct((3, E0), jnp.float32),
    )(nbr_fea.reshape(E0, NF), w1s, b1s, w2s)
    w_all = w_all + b2s.reshape(3, 1)

    idx_flat = nbr_idx.astype(jnp.int32).reshape(E0)
    idx_pad = jnp.pad(idx_flat, (0, APAD * M - E0))
    w_pad = jnp.pad(w_all, ((0, 0), (0, APAD * M - E0)))

    for l, p in enumerate(params["convs"]):
        g_pad = _conv_gather(x, idx_pad, w_pad[l])
        x = pl.pallas_call(
            _bn_body,
            out_shape=jax.ShapeDtypeStruct((N, AF), jnp.float32),
        )(
            x,
            g_pad[:N],
            p["Wtp"],
            p["gamma"].reshape(1, AF),
            p["beta"].reshape(1, AF),
        )

    cidx = crystal_atom_idx.astype(jnp.int32)
    cidx_pad = jnp.pad(cidx, ((0, GPAD - NCRY), (0, APC_PAD - APC)))
    crys = _crystal_pool(x, cidx_pad)[:NCRY]

    out, h = pl.pallas_call(
        _readout_body,
        out_shape=(
            jax.ShapeDtypeStruct((NCRY, 1), jnp.float32),
            jax.ShapeDtypeStruct((NCRY, H), jnp.float32),
        ),
    )(
        crys,
        params["W_fc"],
        params["b_fc"].reshape(1, H),
        params["W_out"],
        params["b_out"].reshape(1, 1),
    )
    return (out, h)


# speedup vs baseline: 3.5569x; 1.3045x over previous
import functools

import jax
import jax.numpy as jnp
from jax import lax
from jax.experimental import pallas as pl
from jax.experimental.pallas import tpu as pltpu
from jax.experimental.pallas import tpu_sc as plsc

N = 10000
M = 32
AF = 128
NF = 16
H = 128
NCRY = 100
APC = 100

NCORE = 2
NSUB = 16
NWORK = NCORE * NSUB
LANES = 16

APW = 320
APAD = NWORK * APW
CA = 4
ECH = CA * M
NCHUNK = APW // CA

GPW = 4
GPAD = NWORK * GPW
APC_PAD = 104

C0 = 0.28209479177387814
EDGE_SCALE = C0 / (float(AF) ** 0.5) / float(M)

_FCH = AF // LANES


def _conv_gather(x, idx_pad, w_pad):
    mesh = plsc.VectorSubcoreMesh(core_axis_name="c", subcore_axis_name="s")
    idx2 = idx_pad.reshape(NWORK * NCHUNK, ECH)
    w2 = w_pad.reshape(NWORK * NCHUNK, ECH)

    @functools.partial(
        pl.kernel,
        out_type=jax.ShapeDtypeStruct((APAD, AF), jnp.float32),
        mesh=mesh,
        scratch_types=[
            pltpu.VMEM((NCHUNK, ECH), jnp.int32),
            pltpu.VMEM((NCHUNK, ECH), jnp.float32),
            pltpu.VMEM((2, ECH, AF), jnp.float32),
            pltpu.VMEM((APW, AF), jnp.float32),
            pltpu.SemaphoreType.DMA,
            pltpu.SemaphoreType.DMA,
        ],
    )
    def k(x_hbm, idx_hbm, w_hbm, out_hbm, idx_all, w_all, rows_v, out_all,
          sem0, sem1):
        wid = lax.axis_index("s") * NCORE + lax.axis_index("c")
        abase = wid * APW
        pltpu.sync_copy(idx_hbm.at[pl.ds(wid * NCHUNK, NCHUNK)], idx_all)
        pltpu.sync_copy(w_hbm.at[pl.ds(wid * NCHUNK, NCHUNK)], w_all)
        sems = (sem0, sem1)

        def fire(c, b):
            pltpu.async_copy(x_hbm.at[idx_all.at[c]], rows_v.at[b], sems[b])

        fire(0, 0)
        fire(1, 1)

        def pair(i, _):
            for b in range(2):
                c = i * 2 + b
                rv = rows_v.at[b]
                pltpu.make_async_copy(
                    x_hbm.at[idx_all.at[c]], rv, sems[b]
                ).wait()

                def atom(a, _):
                    e0 = a * M
                    wv0 = w_all[c, pl.ds(e0, LANES)]
                    wv1 = w_all[c, pl.ds(e0 + LANES, LANES)]
                    accs = [jnp.zeros((LANES,), jnp.float32)
                            for _ in range(_FCH)]
                    for e in range(M):
                        wv = wv0 if e < LANES else wv1
                        lane = jnp.full((LANES,), e % LANES, jnp.int32)
                        wb = wv.at[lane].get(mode="promise_in_bounds")
                        for f in range(_FCH):
                            row = rv[e0 + e, pl.ds(f * LANES, LANES)]
                            accs[f] = accs[f] + wb * row
                    for f in range(_FCH):
                        out_all[c * CA + a, pl.ds(f * LANES, LANES)] = accs[f]
                    return 0

                lax.fori_loop(0, CA, atom, 0)

                nc = c + 2

                @pl.when(nc < NCHUNK)
                def _():
                    fire(nc, b)
            return 0

        lax.fori_loop(0, NCHUNK // 2, pair, 0)
        pltpu.sync_copy(out_all, out_hbm.at[pl.ds(abase, APW)])

    return k(x, idx2, w2)


def _crystal_pool(x, cidx_pad):
    mesh = plsc.VectorSubcoreMesh(core_axis_name="c", subcore_axis_name="s")

    @functools.partial(
        pl.kernel,
        out_type=jax.ShapeDtypeStruct((GPAD, AF), jnp.float32),
        mesh=mesh,
        scratch_types=[
            pltpu.VMEM((GPW, APC_PAD), jnp.int32),
            pltpu.VMEM((2, APC_PAD, AF), jnp.float32),
            pltpu.VMEM((GPW, AF), jnp.float32),
            pltpu.SemaphoreType.DMA,
            pltpu.SemaphoreType.DMA,
        ],
    )
    def k(x_hbm, cidx_hbm, out_hbm, idx_all, rows_v, out_v, sem0, sem1):
        wid = lax.axis_index("s") * NCORE + lax.axis_index("c")
        gbase = wid * GPW
        pltpu.sync_copy(cidx_hbm.at[pl.ds(gbase, GPW)], idx_all)
        sems = (sem0, sem1)

        def fire(g, b):
            pltpu.async_copy(x_hbm.at[idx_all.at[g]], rows_v.at[b], sems[b])

        fire(0, 0)
        fire(1, 1)
        scale = jnp.float32(1.0 / APC)
        for g in range(GPW):
            b = g % 2
            rv = rows_v.at[b]
            pltpu.make_async_copy(x_hbm.at[idx_all.at[g]], rv, sems[b]).wait()
            for f in range(_FCH):
                acc = jnp.zeros((LANES,), jnp.float32)
                for e in range(APC):
                    acc = acc + rv[e, pl.ds(f * LANES, LANES)]
                out_v[g, pl.ds(f * LANES, LANES)] = acc * scale
            if g + 2 < GPW:
                fire(g + 2, b)
        pltpu.sync_copy(out_v, out_hbm.at[pl.ds(gbase, GPW)])

    return k(x, cidx_pad)


def _emb_body(af_ref, w_ref, b_ref, o_ref):
    o_ref[...] = (
        jnp.dot(af_ref[...], w_ref[...], preferred_element_type=jnp.float32)
        + b_ref[...]
    )


def _edge_w_body(nbr_ref, w1_ref, b1_ref, w2_ref, o_ref):
    nbr = nbr_ref[...]
    outs = []
    for l in range(3):
        h = jax.nn.softplus(
            jnp.dot(nbr, w1_ref[l], preferred_element_type=jnp.float32)
            + b1_ref[l]
        )
        s = jnp.dot(h, w2_ref[l], preferred_element_type=jnp.float32)
        outs.append(s.T)
    o_ref[...] = jnp.concatenate(outs, axis=0)


def _mm_body(x_ref, w_ref, o_ref):
    o_ref[...] = jnp.dot(
        x_ref[...], w_ref[...], preferred_element_type=jnp.float32
    )


def _bn_body(x_ref, g_ref, gm_ref, bt_ref, o_ref):
    pre = x_ref[...] + g_ref[...]
    mean = jnp.mean(pre, axis=0, keepdims=True)
    d = pre - mean
    var = jnp.mean(d * d, axis=0, keepdims=True)
    o_ref[...] = jax.nn.softplus(
        d / jnp.sqrt(var + 1e-5) * gm_ref[...] + bt_ref[...]
    )


def _readout_body(c_ref, wfc_ref, bfc_ref, wout_ref, bout_ref, o_ref, h_ref):
    h = jax.nn.softplus(
        jnp.dot(c_ref[...], wfc_ref[...], preferred_element_type=jnp.float32)
        + bfc_ref[...]
    )
    h_ref[...] = h
    o_ref[...] = (
        jnp.dot(h, wout_ref[...], preferred_element_type=jnp.float32)
        + bout_ref[...]
    )


def kernel(atom_fea, nbr_fea, nbr_idx, crystal_atom_idx, pos, params):
    del pos

    x = pl.pallas_call(
        _emb_body,
        out_shape=jax.ShapeDtypeStruct((N, AF), jnp.float32),
    )(atom_fea, params["W_emb"], params["b_emb"].reshape(1, AF))

    w1s = jnp.stack([p["W1"] for p in params["convs"]])
    b1s = jnp.stack([p["b1"].reshape(1, NF) for p in params["convs"]])
    w2s = jnp.stack([p["W2"][:, :1] for p in params["convs"]])
    b2s = jnp.stack([p["b2"][0] for p in params["convs"]])

    E0 = N * M
    EB = 32000
    w_all = pl.pallas_call(
        _edge_w_body,
        grid=(E0 // EB,),
        in_specs=[
            pl.BlockSpec((EB, NF), lambda i: (i, 0)),
            pl.BlockSpec((3, NF, NF), lambda i: (0, 0, 0)),
            pl.BlockSpec((3, 1, NF), lambda i: (0, 0, 0)),
            pl.BlockSpec((3, NF, 1), lambda i: (0, 0, 0)),
        ],
        out_specs=pl.BlockSpec((3, EB), lambda i: (0, i)),
        out_shape=jax.ShapeDtypeStruct((3, E0), jnp.float32),
    )(nbr_fea.reshape(E0, NF), w1s, b1s, w2s)
    w_all = (w_all + b2s.reshape(3, 1)) * EDGE_SCALE

    idx_flat = nbr_idx.astype(jnp.int32).reshape(E0)
    idx_pad = jnp.pad(idx_flat, (0, APAD * M - E0))
    w_pad = jnp.pad(w_all, ((0, 0), (0, APAD * M - E0)))

    for l, p in enumerate(params["convs"]):
        y = pl.pallas_call(
            _mm_body,
            out_shape=jax.ShapeDtypeStruct((N, AF), jnp.float32),
        )(x, p["Wtp"])
        g_pad = _conv_gather(y, idx_pad, w_pad[l])
        x = pl.pallas_call(
            _bn_body,
            out_shape=jax.ShapeDtypeStruct((N, AF), jnp.float32),
        )(
            x,
            g_pad[:N],
            p["gamma"].reshape(1, AF),
            p["beta"].reshape(1, AF),
        )

    cidx = crystal_atom_idx.astype(jnp.int32)
    cidx_pad = jnp.pad(cidx, ((0, GPAD - NCRY), (0, APC_PAD - APC)))
    crys = _crystal_pool(x, cidx_pad)[:NCRY]

    out, h = pl.pallas_call(
        _readout_body,
        out_shape=(
            jax.ShapeDtypeStruct((NCRY, 1), jnp.float32),
            jax.ShapeDtypeStruct((NCRY, H), jnp.float32),
        ),
    )(
        crys,
        params["W_fc"],
        params["b_fc"].reshape(1, H),
        params["W_out"],
        params["b_out"].reshape(1, 1),
    )
    return (out, h)


# speedup vs baseline: 4.1290x; 1.1608x over previous
import functools

import jax
import jax.numpy as jnp
from jax import lax
from jax.experimental import pallas as pl
from jax.experimental.pallas import tpu as pltpu
from jax.experimental.pallas import tpu_sc as plsc

N = 10000
M = 32
AF = 128
NF = 16
H = 128
NCRY = 100
APC = 100

NCORE = 2
NSUB = 16
NWORK = NCORE * NSUB
LANES = 16

APW = 320
APAD = NWORK * APW
CA = 4
ECH = CA * M
NCHUNK = APW // CA

GPW = 4
GPAD = NWORK * GPW
APC_PAD = 104

C0 = 0.28209479177387814
EDGE_SCALE = C0 / (float(AF) ** 0.5) / float(M)

_FCH = AF // LANES


def _conv_gather(x, idx_pad, w_pad):
    mesh = plsc.VectorSubcoreMesh(core_axis_name="c", subcore_axis_name="s")
    idx2 = idx_pad.reshape(NWORK * NCHUNK, ECH)
    w2 = w_pad.reshape(NWORK * NCHUNK, ECH)

    @functools.partial(
        pl.kernel,
        out_type=jax.ShapeDtypeStruct((APAD, AF), jnp.float32),
        mesh=mesh,
        scratch_types=[
            pltpu.VMEM((NCHUNK, ECH), jnp.int32),
            pltpu.VMEM((NCHUNK, ECH), jnp.float32),
            pltpu.VMEM((2, ECH, AF), jnp.float32),
            pltpu.VMEM((APW, AF), jnp.float32),
            pltpu.SemaphoreType.DMA,
            pltpu.SemaphoreType.DMA,
        ],
    )
    def k(x_hbm, idx_hbm, w_hbm, out_hbm, idx_all, w_all, rows_v, out_all,
          sem0, sem1):
        wid = lax.axis_index("s") * NCORE + lax.axis_index("c")
        abase = wid * APW
        pltpu.sync_copy(idx_hbm.at[pl.ds(wid * NCHUNK, NCHUNK)], idx_all)
        pltpu.sync_copy(w_hbm.at[pl.ds(wid * NCHUNK, NCHUNK)], w_all)
        sems = (sem0, sem1)

        def fire(c, b):
            pltpu.async_copy(x_hbm.at[idx_all.at[c]], rows_v.at[b], sems[b])

        fire(0, 0)
        fire(1, 1)

        def pair(i, _):
            for b in range(2):
                c = i * 2 + b
                rv = rows_v.at[b]
                pltpu.make_async_copy(
                    x_hbm.at[idx_all.at[c]], rv, sems[b]
                ).wait()

                def atom(a, _):
                    e0 = a * M
                    wv0 = w_all[c, pl.ds(e0, LANES)]
                    wv1 = w_all[c, pl.ds(e0 + LANES, LANES)]
                    accs = [jnp.zeros((LANES,), jnp.float32)
                            for _ in range(_FCH)]
                    for e in range(M):
                        wv = wv0 if e < LANES else wv1
                        lane = jnp.full((LANES,), e % LANES, jnp.int32)
                        wb = wv.at[lane].get(mode="promise_in_bounds")
                        for f in range(_FCH):
                            row = rv[e0 + e, pl.ds(f * LANES, LANES)]
                            accs[f] = accs[f] + wb * row
                    for f in range(_FCH):
                        out_all[c * CA + a, pl.ds(f * LANES, LANES)] = accs[f]
                    return 0

                lax.fori_loop(0, CA, atom, 0)

                nc = c + 2

                @pl.when(nc < NCHUNK)
                def _():
                    fire(nc, b)
            return 0

        lax.fori_loop(0, NCHUNK // 2, pair, 0)
        pltpu.sync_copy(out_all, out_hbm.at[pl.ds(abase, APW)])

    return k(x, idx2, w2)


def _crystal_pool(x, cidx_pad):
    mesh = plsc.VectorSubcoreMesh(core_axis_name="c", subcore_axis_name="s")

    @functools.partial(
        pl.kernel,
        out_type=jax.ShapeDtypeStruct((GPAD, AF), jnp.float32),
        mesh=mesh,
        scratch_types=[
            pltpu.VMEM((GPW, APC_PAD), jnp.int32),
            pltpu.VMEM((2, APC_PAD, AF), jnp.float32),
            pltpu.VMEM((GPW, AF), jnp.float32),
            pltpu.SemaphoreType.DMA,
            pltpu.SemaphoreType.DMA,
        ],
    )
    def k(x_hbm, cidx_hbm, out_hbm, idx_all, rows_v, out_v, sem0, sem1):
        wid = lax.axis_index("s") * NCORE + lax.axis_index("c")
        gbase = wid * GPW
        pltpu.sync_copy(cidx_hbm.at[pl.ds(gbase, GPW)], idx_all)
        sems = (sem0, sem1)

        def fire(g, b):
            pltpu.async_copy(x_hbm.at[idx_all.at[g]], rows_v.at[b], sems[b])

        fire(0, 0)
        fire(1, 1)
        scale = jnp.float32(1.0 / APC)
        for g in range(GPW):
            b = g % 2
            rv = rows_v.at[b]
            pltpu.make_async_copy(x_hbm.at[idx_all.at[g]], rv, sems[b]).wait()
            for f in range(_FCH):
                acc = jnp.zeros((LANES,), jnp.float32)
                for e in range(APC):
                    acc = acc + rv[e, pl.ds(f * LANES, LANES)]
                out_v[g, pl.ds(f * LANES, LANES)] = acc * scale
            if g + 2 < GPW:
                fire(g + 2, b)
        pltpu.sync_copy(out_v, out_hbm.at[pl.ds(gbase, GPW)])

    return k(x, cidx_pad)


def _emb_body(af_ref, w_ref, b_ref, o_ref):
    o_ref[...] = (
        jnp.dot(af_ref[...], w_ref[...], preferred_element_type=jnp.float32)
        + b_ref[...]
    )


def _edge_w_body(nbrT_ref, w1t_ref, b1_ref, w2t_ref, o_ref):
    nbrT = nbrT_ref[...]
    outs = []
    for l in range(3):
        hT = jax.nn.softplus(
            jnp.dot(w1t_ref[l], nbrT, preferred_element_type=jnp.float32)
            + b1_ref[l]
        )
        sT = jnp.dot(w2t_ref[l], hT, preferred_element_type=jnp.float32)
        outs.append(sT)
    o_ref[...] = jnp.concatenate(outs, axis=0)


def _mm_body(x_ref, w_ref, o_ref):
    o_ref[...] = jnp.dot(
        x_ref[...], w_ref[...], preferred_element_type=jnp.float32
    )


def _bn_body(x_ref, g_ref, gm_ref, bt_ref, o_ref):
    pre = x_ref[...] + g_ref[...]
    mean = jnp.mean(pre, axis=0, keepdims=True)
    d = pre - mean
    var = jnp.mean(d * d, axis=0, keepdims=True)
    o_ref[...] = jax.nn.softplus(
        d / jnp.sqrt(var + 1e-5) * gm_ref[...] + bt_ref[...]
    )


def _readout_body(c_ref, wfc_ref, bfc_ref, wout_ref, bout_ref, o_ref, h_ref):
    h = jax.nn.softplus(
        jnp.dot(c_ref[...], wfc_ref[...], preferred_element_type=jnp.float32)
        + bfc_ref[...]
    )
    h_ref[...] = h
    o_ref[...] = (
        jnp.dot(h, wout_ref[...], preferred_element_type=jnp.float32)
        + bout_ref[...]
    )


def kernel(atom_fea, nbr_fea, nbr_idx, crystal_atom_idx, pos, params):
    del pos

    x = pl.pallas_call(
        _emb_body,
        out_shape=jax.ShapeDtypeStruct((N, AF), jnp.float32),
    )(atom_fea, params["W_emb"], params["b_emb"].reshape(1, AF))

    w1ts = jnp.stack([p["W1"].T for p in params["convs"]])
    b1s = jnp.stack([p["b1"].reshape(NF, 1) for p in params["convs"]])
    w2ts = jnp.stack([p["W2"][:, :1].T for p in params["convs"]])
    b2s = jnp.stack([p["b2"][0] for p in params["convs"]])

    E0 = N * M
    EB = 32000
    w_all = pl.pallas_call(
        _edge_w_body,
        grid=(E0 // EB,),
        in_specs=[
            pl.BlockSpec((NF, EB), lambda i: (0, i)),
            pl.BlockSpec((3, NF, NF), lambda i: (0, 0, 0)),
            pl.BlockSpec((3, NF, 1), lambda i: (0, 0, 0)),
            pl.BlockSpec((3, 1, NF), lambda i: (0, 0, 0)),
        ],
        out_specs=pl.BlockSpec((3, EB), lambda i: (0, i)),
        out_shape=jax.ShapeDtypeStruct((3, E0), jnp.float32),
    )(nbr_fea.reshape(E0, NF).T, w1ts, b1s, w2ts)
    w_all = (w_all + b2s.reshape(3, 1)) * EDGE_SCALE

    idx_flat = nbr_idx.astype(jnp.int32).reshape(E0)
    idx_pad = jnp.pad(idx_flat, (0, APAD * M - E0))
    w_pad = jnp.pad(w_all, ((0, 0), (0, APAD * M - E0)))

    for l, p in enumerate(params["convs"]):
        y = pl.pallas_call(
            _mm_body,
            out_shape=jax.ShapeDtypeStruct((N, AF), jnp.float32),
        )(x, p["Wtp"])
        g_pad = _conv_gather(y, idx_pad, w_pad[l])
        x = pl.pallas_call(
            _bn_body,
            out_shape=jax.ShapeDtypeStruct((N, AF), jnp.float32),
        )(
            x,
            g_pad[:N],
            p["gamma"].reshape(1, AF),
            p["beta"].reshape(1, AF),
        )

    cidx = crystal_atom_idx.astype(jnp.int32)
    cidx_pad = jnp.pad(cidx, ((0, GPAD - NCRY), (0, APC_PAD - APC)))
    crys = _crystal_pool(x, cidx_pad)[:NCRY]

    out, h = pl.pallas_call(
        _readout_body,
        out_shape=(
            jax.ShapeDtypeStruct((NCRY, 1), jnp.float32),
            jax.ShapeDtypeStruct((NCRY, H), jnp.float32),
        ),
    )(
        crys,
        params["W_fc"],
        params["b_fc"].reshape(1, H),
        params["W_out"],
        params["b_out"].reshape(1, 1),
    )
    return (out, h)


# speedup vs baseline: 4.2159x; 1.0211x over previous
import functools

import jax
import jax.numpy as jnp
from jax import lax
from jax.experimental import pallas as pl
from jax.experimental.pallas import tpu as pltpu
from jax.experimental.pallas import tpu_sc as plsc

N = 10000
M = 32
AF = 128
NF = 16
H = 128
NCRY = 100
APC = 100

NCORE = 2
NSUB = 16
NWORK = NCORE * NSUB
LANES = 16

FAST_C = 0
FA = 480
SL = 160
APAD = NSUB * (FA + SL)
CA = 2
ECH = CA * M
NCF = FA // CA
NCS = SL // CA
TOTAL_CHUNKS = APAD * M // ECH

GPW = 4
GPAD = NWORK * GPW
APC_PAD = 104

C0 = 0.28209479177387814
EDGE_SCALE = C0 / (float(AF) ** 0.5) / float(M)

_FCH = AF // LANES


def _conv_gather(x, idx_pad, w_pad):
    mesh = plsc.VectorSubcoreMesh(core_axis_name="c", subcore_axis_name="s")
    idx2 = idx_pad.reshape(TOTAL_CHUNKS, ECH)
    w2 = w_pad.reshape(TOTAL_CHUNKS, ECH)

    sz_even, sz_odd = (FA, SL) if FAST_C == 0 else (SL, FA)

    @functools.partial(
        pl.kernel,
        out_type=jax.ShapeDtypeStruct((APAD, AF), jnp.float32),
        mesh=mesh,
        scratch_types=[
            pltpu.VMEM((NCF, ECH), jnp.int32),
            pltpu.VMEM((NCF, ECH), jnp.float32),
            pltpu.VMEM((2, ECH, AF), jnp.float32),
            pltpu.VMEM((FA // 2, AF), jnp.float32),
            pltpu.SemaphoreType.DMA,
            pltpu.SemaphoreType.DMA,
        ],
    )
    def k(x_hbm, idx_hbm, w_hbm, out_hbm, idx_all, w_all, rows_v, out_all,
          sem0, sem1):
        cc = lax.axis_index("c")
        wid = lax.axis_index("s") * NCORE + cc
        n_even = (wid + 1) // 2
        n_odd = wid // 2
        abase = pl.multiple_of(n_even * sz_even + n_odd * sz_odd, 16)
        cbase = pl.multiple_of(abase // CA, 8)
        is_fast = cc == FAST_C
        my_nchunk = jnp.where(is_fast, NCF, NCS)

        @pl.when(is_fast)
        def _():
            pltpu.sync_copy(idx_hbm.at[pl.ds(cbase, NCF)], idx_all)
            pltpu.sync_copy(w_hbm.at[pl.ds(cbase, NCF)], w_all)

        @pl.when(jnp.logical_not(is_fast))
        def _():
            pltpu.sync_copy(idx_hbm.at[pl.ds(cbase, NCS)],
                            idx_all.at[pl.ds(0, NCS)])
            pltpu.sync_copy(w_hbm.at[pl.ds(cbase, NCS)],
                            w_all.at[pl.ds(0, NCS)])

        sems = (sem0, sem1)

        def fire(c, b):
            pltpu.async_copy(x_hbm.at[idx_all.at[c]], rows_v.at[b], sems[b])

        PH = NCF // 2
        for p in range(2):
            if p == 0:
                ph_n = jnp.where(is_fast, PH, NCS)
            else:
                ph_n = jnp.where(is_fast, PH, 0)
            ph_base = p * PH

            @pl.when(ph_n > 0)
            def _(ph_n=ph_n, ph_base=ph_base, p=p):
                fire(ph_base, 0)
                fire(ph_base + 1, 1)

                def pair(i, _):
                    for b in range(2):
                        c = ph_base + i * 2 + b
                        rv = rows_v.at[b]
                        pltpu.make_async_copy(
                            x_hbm.at[idx_all.at[c]], rv, sems[b]
                        ).wait()

                        def atom(a, _):
                            e0 = a * M
                            wv0 = w_all[c, pl.ds(e0, LANES)]
                            wv1 = w_all[c, pl.ds(e0 + LANES, LANES)]
                            accs = [jnp.zeros((LANES,), jnp.float32)
                                    for _ in range(_FCH)]
                            for e in range(M):
                                wv = wv0 if e < LANES else wv1
                                lane = jnp.full((LANES,), e % LANES,
                                                jnp.int32)
                                wb = wv.at[lane].get(
                                    mode="promise_in_bounds")
                                for f in range(_FCH):
                                    row = rv[e0 + e,
                                             pl.ds(f * LANES, LANES)]
                                    accs[f] = accs[f] + wb * row
                            lrow = (c - ph_base) * CA + a
                            for f in range(_FCH):
                                out_all[lrow, pl.ds(f * LANES, LANES)] = (
                                    accs[f])
                            return 0

                        lax.fori_loop(0, CA, atom, 0)

                        nc = c + 2

                        @pl.when(nc < ph_base + ph_n)
                        def _():
                            fire(nc, b)
                    return 0

                lax.fori_loop(0, ph_n // 2, pair, 0)

                obase = pl.multiple_of(abase + ph_base * CA, 16)

                @pl.when(is_fast)
                def _():
                    pltpu.sync_copy(out_all,
                                    out_hbm.at[pl.ds(obase, FA // 2)])

                @pl.when(jnp.logical_not(is_fast))
                def _():
                    pltpu.sync_copy(out_all.at[pl.ds(0, SL)],
                                    out_hbm.at[pl.ds(obase, SL)])

    return k(x, idx2, w2)


def _crystal_pool(x, cidx_pad):
    mesh = plsc.VectorSubcoreMesh(core_axis_name="c", subcore_axis_name="s")

    @functools.partial(
        pl.kernel,
        out_type=jax.ShapeDtypeStruct((GPAD, AF), jnp.float32),
        mesh=mesh,
        scratch_types=[
            pltpu.VMEM((GPW, APC_PAD), jnp.int32),
            pltpu.VMEM((2, APC_PAD, AF), jnp.float32),
            pltpu.VMEM((GPW, AF), jnp.float32),
            pltpu.SemaphoreType.DMA,
            pltpu.SemaphoreType.DMA,
        ],
    )
    def k(x_hbm, cidx_hbm, out_hbm, idx_all, rows_v, out_v, sem0, sem1):
        wid = lax.axis_index("s") * NCORE + lax.axis_index("c")
        gbase = wid * GPW
        pltpu.sync_copy(cidx_hbm.at[pl.ds(gbase, GPW)], idx_all)
        sems = (sem0, sem1)

        def fire(g, b):
            pltpu.async_copy(x_hbm.at[idx_all.at[g]], rows_v.at[b], sems[b])

        fire(0, 0)
        fire(1, 1)
        scale = jnp.float32(1.0 / APC)
        for g in range(GPW):
            b = g % 2
            rv = rows_v.at[b]
            pltpu.make_async_copy(x_hbm.at[idx_all.at[g]], rv, sems[b]).wait()
            for f in range(_FCH):
                acc = jnp.zeros((LANES,), jnp.float32)
                for e in range(APC):
                    acc = acc + rv[e, pl.ds(f * LANES, LANES)]
                out_v[g, pl.ds(f * LANES, LANES)] = acc * scale
            if g + 2 < GPW:
                fire(g + 2, b)
        pltpu.sync_copy(out_v, out_hbm.at[pl.ds(gbase, GPW)])

    return k(x, cidx_pad)


def _emb_body(af_ref, w_ref, b_ref, o_ref):
    o_ref[...] = (
        jnp.dot(af_ref[...], w_ref[...], preferred_element_type=jnp.float32)
        + b_ref[...]
    )


def _edge_w_body(nbrT_ref, w1t_ref, b1_ref, w2t_ref, o_ref):
    nbrT = nbrT_ref[...]
    outs = []
    for l in range(3):
        hT = jax.nn.softplus(
            jnp.dot(w1t_ref[l], nbrT, preferred_element_type=jnp.float32)
            + b1_ref[l]
        )
        sT = jnp.dot(w2t_ref[l], hT, preferred_element_type=jnp.float32)
        outs.append(sT)
    o_ref[...] = jnp.concatenate(outs, axis=0)


def _mm_body(x_ref, w_ref, o_ref):
    o_ref[...] = jnp.dot(
        x_ref[...], w_ref[...], preferred_element_type=jnp.float32
    )


def _bn_body(x_ref, g_ref, gm_ref, bt_ref, o_ref):
    pre = x_ref[...] + g_ref[...]
    mean = jnp.mean(pre, axis=0, keepdims=True)
    d = pre - mean
    var = jnp.mean(d * d, axis=0, keepdims=True)
    o_ref[...] = jax.nn.softplus(
        d / jnp.sqrt(var + 1e-5) * gm_ref[...] + bt_ref[...]
    )


def _readout_body(c_ref, wfc_ref, bfc_ref, wout_ref, bout_ref, o_ref, h_ref):
    h = jax.nn.softplus(
        jnp.dot(c_ref[...], wfc_ref[...], preferred_element_type=jnp.float32)
        + bfc_ref[...]
    )
    h_ref[...] = h
    o_ref[...] = (
        jnp.dot(h, wout_ref[...], preferred_element_type=jnp.float32)
        + bout_ref[...]
    )


def kernel(atom_fea, nbr_fea, nbr_idx, crystal_atom_idx, pos, params):
    del pos

    x = pl.pallas_call(
        _emb_body,
        out_shape=jax.ShapeDtypeStruct((N, AF), jnp.float32),
    )(atom_fea, params["W_emb"], params["b_emb"].reshape(1, AF))

    w1ts = jnp.stack([p["W1"].T for p in params["convs"]])
    b1s = jnp.stack([p["b1"].reshape(NF, 1) for p in params["convs"]])
    w2ts = jnp.stack([p["W2"][:, :1].T for p in params["convs"]])
    b2s = jnp.stack([p["b2"][0] for p in params["convs"]])

    E0 = N * M
    EB = 32000
    w_all = pl.pallas_call(
        _edge_w_body,
        grid=(E0 // EB,),
        in_specs=[
            pl.BlockSpec((NF, EB), lambda i: (0, i)),
            pl.BlockSpec((3, NF, NF), lambda i: (0, 0, 0)),
            pl.BlockSpec((3, NF, 1), lambda i: (0, 0, 0)),
            pl.BlockSpec((3, 1, NF), lambda i: (0, 0, 0)),
        ],
        out_specs=pl.BlockSpec((3, EB), lambda i: (0, i)),
        out_shape=jax.ShapeDtypeStruct((3, E0), jnp.float32),
    )(nbr_fea.reshape(E0, NF).T, w1ts, b1s, w2ts)
    w_all = (w_all + b2s.reshape(3, 1)) * EDGE_SCALE

    idx_flat = nbr_idx.astype(jnp.int32).reshape(E0)
    idx_pad = jnp.pad(idx_flat, (0, APAD * M - E0))
    w_pad = jnp.pad(w_all, ((0, 0), (0, APAD * M - E0)))

    for l, p in enumerate(params["convs"]):
        y = pl.pallas_call(
            _mm_body,
            out_shape=jax.ShapeDtypeStruct((N, AF), jnp.float32),
        )(x, p["Wtp"])
        g_pad = _conv_gather(y, idx_pad, w_pad[l])
        x = pl.pallas_call(
            _bn_body,
            out_shape=jax.ShapeDtypeStruct((N, AF), jnp.float32),
        )(
            x,
            g_pad[:N],
            p["gamma"].reshape(1, AF),
            p["beta"].reshape(1, AF),
        )

    cidx = crystal_atom_idx.astype(jnp.int32)
    cidx_pad = jnp.pad(cidx, ((0, GPAD - NCRY), (0, APC_PAD - APC)))
    crys = _crystal_pool(x, cidx_pad)[:NCRY]

    out, h = pl.pallas_call(
        _readout_body,
        out_shape=(
            jax.ShapeDtypeStruct((NCRY, 1), jnp.float32),
            jax.ShapeDtypeStruct((NCRY, H), jnp.float32),
        ),
    )(
        crys,
        params["W_fc"],
        params["b_fc"].reshape(1, H),
        params["W_out"],
        params["b_out"].reshape(1, 1),
    )
    return (out, h)


# speedup vs baseline: 4.2200x; 1.0010x over previous
import functools

import jax
import jax.numpy as jnp
from jax import lax
from jax.experimental import pallas as pl
from jax.experimental.pallas import tpu as pltpu
from jax.experimental.pallas import tpu_sc as plsc

N = 10000
M = 32
AF = 128
NF = 16
H = 128
NCRY = 100
APC = 100

NCORE = 2
NSUB = 16
NWORK = NCORE * NSUB
LANES = 16

FAST_C = 0
FA = 480
SL = 160
APAD = NSUB * (FA + SL)
CA = 2
ECH = CA * M
NCF = FA // CA
NCS = SL // CA
TOTAL_CHUNKS = APAD * M // ECH

GPW = 4
GPAD = NWORK * GPW
APC_PAD = 104

C0 = 0.28209479177387814
EDGE_SCALE = C0 / (float(AF) ** 0.5) / float(M)

_FCH = AF // LANES


def _conv_gather(x, idx_pad, w_pad):
    mesh = plsc.VectorSubcoreMesh(core_axis_name="c", subcore_axis_name="s")
    idx2 = idx_pad.reshape(TOTAL_CHUNKS, ECH)
    w2 = w_pad.reshape(TOTAL_CHUNKS, ECH)

    sz_even, sz_odd = (FA, SL) if FAST_C == 0 else (SL, FA)

    @functools.partial(
        pl.kernel,
        out_type=jax.ShapeDtypeStruct((APAD, AF), jnp.float32),
        mesh=mesh,
        scratch_types=[
            pltpu.VMEM((NCF, ECH), jnp.int32),
            pltpu.VMEM((NCF, ECH), jnp.float32),
            pltpu.VMEM((4, ECH, AF), jnp.float32),
            pltpu.VMEM((FA // 2, AF), jnp.float32),
            pltpu.SemaphoreType.DMA,
            pltpu.SemaphoreType.DMA,
            pltpu.SemaphoreType.DMA,
            pltpu.SemaphoreType.DMA,
        ],
    )
    def k(x_hbm, idx_hbm, w_hbm, out_hbm, idx_all, w_all, rows_v, out_all,
          sem0, sem1, sem2, sem3):
        cc = lax.axis_index("c")
        wid = lax.axis_index("s") * NCORE + cc
        n_even = (wid + 1) // 2
        n_odd = wid // 2
        abase = pl.multiple_of(n_even * sz_even + n_odd * sz_odd, 16)
        cbase = pl.multiple_of(abase // CA, 8)
        is_fast = cc == FAST_C
        my_nchunk = jnp.where(is_fast, NCF, NCS)

        @pl.when(is_fast)
        def _():
            pltpu.sync_copy(idx_hbm.at[pl.ds(cbase, NCF)], idx_all)
            pltpu.sync_copy(w_hbm.at[pl.ds(cbase, NCF)], w_all)

        @pl.when(jnp.logical_not(is_fast))
        def _():
            pltpu.sync_copy(idx_hbm.at[pl.ds(cbase, NCS)],
                            idx_all.at[pl.ds(0, NCS)])
            pltpu.sync_copy(w_hbm.at[pl.ds(cbase, NCS)],
                            w_all.at[pl.ds(0, NCS)])

        sems = (sem0, sem1, sem2, sem3)

        def fire(c, b):
            pltpu.async_copy(x_hbm.at[idx_all.at[c]], rows_v.at[b], sems[b])

        PH = NCF // 2
        for p in range(2):
            if p == 0:
                ph_n = jnp.where(is_fast, PH, NCS)
            else:
                ph_n = jnp.where(is_fast, PH, 0)
            ph_base = p * PH

            @pl.when(ph_n > 0)
            def _(ph_n=ph_n, ph_base=ph_base, p=p):
                for b in range(4):
                    fire(ph_base + b, b)

                def pair(i, _):
                    for b in range(4):
                        c = ph_base + i * 4 + b
                        rv = rows_v.at[b]
                        pltpu.make_async_copy(
                            x_hbm.at[idx_all.at[c]], rv, sems[b]
                        ).wait()

                        def atom(a, _):
                            e0 = a * M
                            wv0 = w_all[c, pl.ds(e0, LANES)]
                            wv1 = w_all[c, pl.ds(e0 + LANES, LANES)]
                            accs = [jnp.zeros((LANES,), jnp.float32)
                                    for _ in range(_FCH)]
                            for e in range(M):
                                wv = wv0 if e < LANES else wv1
                                lane = jnp.full((LANES,), e % LANES,
                                                jnp.int32)
                                wb = wv.at[lane].get(
                                    mode="promise_in_bounds")
                                for f in range(_FCH):
                                    row = rv[e0 + e,
                                             pl.ds(f * LANES, LANES)]
                                    accs[f] = accs[f] + wb * row
                            lrow = (c - ph_base) * CA + a
                            for f in range(_FCH):
                                out_all[lrow, pl.ds(f * LANES, LANES)] = (
                                    accs[f])
                            return 0

                        lax.fori_loop(0, CA, atom, 0)

                        nc = c + 4

                        @pl.when(nc < ph_base + ph_n)
                        def _():
                            fire(nc, b)
                    return 0

                lax.fori_loop(0, ph_n // 4, pair, 0)

                obase = pl.multiple_of(abase + ph_base * CA, 16)

                @pl.when(is_fast)
                def _():
                    pltpu.sync_copy(out_all,
                                    out_hbm.at[pl.ds(obase, FA // 2)])

                @pl.when(jnp.logical_not(is_fast))
                def _():
                    pltpu.sync_copy(out_all.at[pl.ds(0, SL)],
                                    out_hbm.at[pl.ds(obase, SL)])

    return k(x, idx2, w2)


def _crystal_pool(x, cidx_pad):
    mesh = plsc.VectorSubcoreMesh(core_axis_name="c", subcore_axis_name="s")

    @functools.partial(
        pl.kernel,
        out_type=jax.ShapeDtypeStruct((GPAD, AF), jnp.float32),
        mesh=mesh,
        scratch_types=[
            pltpu.VMEM((GPW, APC_PAD), jnp.int32),
            pltpu.VMEM((2, APC_PAD, AF), jnp.float32),
            pltpu.VMEM((GPW, AF), jnp.float32),
            pltpu.SemaphoreType.DMA,
            pltpu.SemaphoreType.DMA,
        ],
    )
    def k(x_hbm, cidx_hbm, out_hbm, idx_all, rows_v, out_v, sem0, sem1):
        wid = lax.axis_index("s") * NCORE + lax.axis_index("c")
        gbase = wid * GPW
        pltpu.sync_copy(cidx_hbm.at[pl.ds(gbase, GPW)], idx_all)
        sems = (sem0, sem1)

        def fire(g, b):
            pltpu.async_copy(x_hbm.at[idx_all.at[g]], rows_v.at[b], sems[b])

        fire(0, 0)
        fire(1, 1)
        scale = jnp.float32(1.0 / APC)
        for g in range(GPW):
            b = g % 2
            rv = rows_v.at[b]
            pltpu.make_async_copy(x_hbm.at[idx_all.at[g]], rv, sems[b]).wait()
            for f in range(_FCH):
                acc = jnp.zeros((LANES,), jnp.float32)
                for e in range(APC):
                    acc = acc + rv[e, pl.ds(f * LANES, LANES)]
                out_v[g, pl.ds(f * LANES, LANES)] = acc * scale
            if g + 2 < GPW:
                fire(g + 2, b)
        pltpu.sync_copy(out_v, out_hbm.at[pl.ds(gbase, GPW)])

    return k(x, cidx_pad)


def _emb_body(af_ref, w_ref, b_ref, o_ref):
    o_ref[...] = (
        jnp.dot(af_ref[...], w_ref[...], preferred_element_type=jnp.float32)
        + b_ref[...]
    )


def _edge_w_body(nbrT_ref, w1t_ref, b1_ref, w2t_ref, o_ref):
    nbrT = nbrT_ref[...]
    outs = []
    for l in range(3):
        hT = jax.nn.softplus(
            jnp.dot(w1t_ref[l], nbrT, preferred_element_type=jnp.float32)
            + b1_ref[l]
        )
        sT = jnp.dot(w2t_ref[l], hT, preferred_element_type=jnp.float32)
        outs.append(sT)
    o_ref[...] = jnp.concatenate(outs, axis=0)


def _mm_body(x_ref, w_ref, o_ref):
    o_ref[...] = jnp.dot(
        x_ref[...], w_ref[...], preferred_element_type=jnp.float32
    )


def _bn_body(x_ref, g_ref, gm_ref, bt_ref, o_ref):
    pre = x_ref[...] + g_ref[...]
    mean = jnp.mean(pre, axis=0, keepdims=True)
    d = pre - mean
    var = jnp.mean(d * d, axis=0, keepdims=True)
    o_ref[...] = jax.nn.softplus(
        d / jnp.sqrt(var + 1e-5) * gm_ref[...] + bt_ref[...]
    )


def _readout_body(c_ref, wfc_ref, bfc_ref, wout_ref, bout_ref, o_ref, h_ref):
    h = jax.nn.softplus(
        jnp.dot(c_ref[...], wfc_ref[...], preferred_element_type=jnp.float32)
        + bfc_ref[...]
    )
    h_ref[...] = h
    o_ref[...] = (
        jnp.dot(h, wout_ref[...], preferred_element_type=jnp.float32)
        + bout_ref[...]
    )


def kernel(atom_fea, nbr_fea, nbr_idx, crystal_atom_idx, pos, params):
    del pos

    x = pl.pallas_call(
        _emb_body,
        out_shape=jax.ShapeDtypeStruct((N, AF), jnp.float32),
    )(atom_fea, params["W_emb"], params["b_emb"].reshape(1, AF))

    w1ts = jnp.stack([p["W1"].T for p in params["convs"]])
    b1s = jnp.stack([p["b1"].reshape(NF, 1) for p in params["convs"]])
    w2ts = jnp.stack([p["W2"][:, :1].T for p in params["convs"]])
    b2s = jnp.stack([p["b2"][0] for p in params["convs"]])

    E0 = N * M
    EB = 32000
    w_all = pl.pallas_call(
        _edge_w_body,
        grid=(E0 // EB,),
        in_specs=[
            pl.BlockSpec((NF, EB), lambda i: (0, i)),
            pl.BlockSpec((3, NF, NF), lambda i: (0, 0, 0)),
            pl.BlockSpec((3, NF, 1), lambda i: (0, 0, 0)),
            pl.BlockSpec((3, 1, NF), lambda i: (0, 0, 0)),
        ],
        out_specs=pl.BlockSpec((3, EB), lambda i: (0, i)),
        out_shape=jax.ShapeDtypeStruct((3, E0), jnp.float32),
    )(nbr_fea.reshape(E0, NF).T, w1ts, b1s, w2ts)
    w_all = (w_all + b2s.reshape(3, 1)) * EDGE_SCALE

    idx_flat = nbr_idx.astype(jnp.int32).reshape(E0)
    idx_pad = jnp.pad(idx_flat, (0, APAD * M - E0))
    w_pad = jnp.pad(w_all, ((0, 0), (0, APAD * M - E0)))

    for l, p in enumerate(params["convs"]):
        y = pl.pallas_call(
            _mm_body,
            out_shape=jax.ShapeDtypeStruct((N, AF), jnp.float32),
        )(x, p["Wtp"])
        g_pad = _conv_gather(y, idx_pad, w_pad[l])
        x = pl.pallas_call(
            _bn_body,
            out_shape=jax.ShapeDtypeStruct((N, AF), jnp.float32),
        )(
            x,
            g_pad[:N],
            p["gamma"].reshape(1, AF),
            p["beta"].reshape(1, AF),
        )

    cidx = crystal_atom_idx.astype(jnp.int32)
    cidx_pad = jnp.pad(cidx, ((0, GPAD - NCRY), (0, APC_PAD - APC)))
    crys = _crystal_pool(x, cidx_pad)[:NCRY]

    out, h = pl.pallas_call(
        _readout_body,
        out_shape=(
            jax.ShapeDtypeStruct((NCRY, 1), jnp.float32),
            jax.ShapeDtypeStruct((NCRY, H), jnp.float32),
        ),
    )(
        crys,
        params["W_fc"],
        params["b_fc"].reshape(1, H),
        params["W_out"],
        params["b_out"].reshape(1, 1),
    )
    return (out, h)


# speedup vs baseline: 4.3273x; 1.0254x over previous
import functools

import jax
import jax.numpy as jnp
from jax import lax
from jax.experimental import pallas as pl
from jax.experimental.pallas import tpu as pltpu
from jax.experimental.pallas import tpu_sc as plsc

N = 10000
M = 32
AF = 128
NF = 16
H = 128
NCRY = 100
APC = 100

NCORE = 2
NSUB = 16
NWORK = NCORE * NSUB
LANES = 16

FAST_C = 0
FA = 480
SL = 160
APAD = NSUB * (FA + SL)
CA = 4
ECH = CA * M
NCF = FA // CA
NCS = SL // CA
TOTAL_CHUNKS = APAD * M // ECH

GPW = 4
GPAD = NWORK * GPW
APC_PAD = 104

C0 = 0.28209479177387814
EDGE_SCALE = C0 / (float(AF) ** 0.5) / float(M)

_FCH = AF // LANES


def _conv_gather(x, idx_pad, w_pad):
    mesh = plsc.VectorSubcoreMesh(core_axis_name="c", subcore_axis_name="s")
    idx2 = idx_pad.reshape(TOTAL_CHUNKS, ECH)
    w2 = w_pad.reshape(TOTAL_CHUNKS, ECH)

    sz_even, sz_odd = (FA, SL) if FAST_C == 0 else (SL, FA)

    @functools.partial(
        pl.kernel,
        out_type=jax.ShapeDtypeStruct((APAD, AF), jnp.float32),
        mesh=mesh,
        scratch_types=[
            pltpu.VMEM((NCF, ECH), jnp.int32),
            pltpu.VMEM((NCF, ECH), jnp.float32),
            pltpu.VMEM((2, ECH, AF), jnp.float32),
            pltpu.VMEM((FA // 2, AF), jnp.float32),
            pltpu.SemaphoreType.DMA,
            pltpu.SemaphoreType.DMA,
        ],
    )
    def k(x_hbm, idx_hbm, w_hbm, out_hbm, idx_all, w_all, rows_v, out_all,
          sem0, sem1):
        cc = lax.axis_index("c")
        wid = lax.axis_index("s") * NCORE + cc
        n_even = (wid + 1) // 2
        n_odd = wid // 2
        abase = pl.multiple_of(n_even * sz_even + n_odd * sz_odd, 16)
        cbase = pl.multiple_of(abase // CA, 8)
        is_fast = cc == FAST_C
        my_nchunk = jnp.where(is_fast, NCF, NCS)

        @pl.when(is_fast)
        def _():
            pltpu.sync_copy(idx_hbm.at[pl.ds(cbase, NCF)], idx_all)
            pltpu.sync_copy(w_hbm.at[pl.ds(cbase, NCF)], w_all)

        @pl.when(jnp.logical_not(is_fast))
        def _():
            pltpu.sync_copy(idx_hbm.at[pl.ds(cbase, NCS)],
                            idx_all.at[pl.ds(0, NCS)])
            pltpu.sync_copy(w_hbm.at[pl.ds(cbase, NCS)],
                            w_all.at[pl.ds(0, NCS)])

        sems = (sem0, sem1)

        def fire(c, b):
            pltpu.async_copy(x_hbm.at[idx_all.at[c]], rows_v.at[b], sems[b])

        PH = NCF // 2
        for p in range(2):
            if p == 0:
                ph_n = jnp.where(is_fast, PH, NCS)
            else:
                ph_n = jnp.where(is_fast, PH, 0)
            ph_base = p * PH

            @pl.when(ph_n > 0)
            def _(ph_n=ph_n, ph_base=ph_base, p=p):
                for b in range(2):
                    fire(ph_base + b, b)

                def pair(i, _):
                    for b in range(2):
                        c = ph_base + i * 2 + b
                        rv = rows_v.at[b]
                        pltpu.make_async_copy(
                            x_hbm.at[idx_all.at[c]], rv, sems[b]
                        ).wait()

                        def atom(a, _):
                            e0 = a * M
                            wv0 = w_all[c, pl.ds(e0, LANES)]
                            wv1 = w_all[c, pl.ds(e0 + LANES, LANES)]
                            accs = [jnp.zeros((LANES,), jnp.float32)
                                    for _ in range(_FCH)]
                            for e in range(M):
                                wv = wv0 if e < LANES else wv1
                                lane = jnp.full((LANES,), e % LANES,
                                                jnp.int32)
                                wb = wv.at[lane].get(
                                    mode="promise_in_bounds")
                                for f in range(_FCH):
                                    row = rv[e0 + e,
                                             pl.ds(f * LANES, LANES)]
                                    accs[f] = accs[f] + wb * row
                            lrow = (c - ph_base) * CA + a
                            for f in range(_FCH):
                                out_all[lrow, pl.ds(f * LANES, LANES)] = (
                                    accs[f])
                            return 0

                        lax.fori_loop(0, CA, atom, 0)

                        nc = c + 2

                        @pl.when(nc < ph_base + ph_n)
                        def _():
                            fire(nc, b)
                    return 0

                lax.fori_loop(0, ph_n // 2, pair, 0)

                obase = pl.multiple_of(abase + ph_base * CA, 16)

                @pl.when(is_fast)
                def _():
                    pltpu.sync_copy(out_all,
                                    out_hbm.at[pl.ds(obase, FA // 2)])

                @pl.when(jnp.logical_not(is_fast))
                def _():
                    pltpu.sync_copy(out_all.at[pl.ds(0, SL)],
                                    out_hbm.at[pl.ds(obase, SL)])

    return k(x, idx2, w2)


def _crystal_pool(x, cidx_pad):
    mesh = plsc.VectorSubcoreMesh(core_axis_name="c", subcore_axis_name="s")

    @functools.partial(
        pl.kernel,
        out_type=jax.ShapeDtypeStruct((GPAD, AF), jnp.float32),
        mesh=mesh,
        scratch_types=[
            pltpu.VMEM((GPW, APC_PAD), jnp.int32),
            pltpu.VMEM((2, APC_PAD, AF), jnp.float32),
            pltpu.VMEM((GPW, AF), jnp.float32),
            pltpu.SemaphoreType.DMA,
            pltpu.SemaphoreType.DMA,
        ],
    )
    def k(x_hbm, cidx_hbm, out_hbm, idx_all, rows_v, out_v, sem0, sem1):
        wid = lax.axis_index("s") * NCORE + lax.axis_index("c")
        gbase = wid * GPW
        pltpu.sync_copy(cidx_hbm.at[pl.ds(gbase, GPW)], idx_all)
        sems = (sem0, sem1)

        def fire(g, b):
            pltpu.async_copy(x_hbm.at[idx_all.at[g]], rows_v.at[b], sems[b])

        fire(0, 0)
        fire(1, 1)
        scale = jnp.float32(1.0 / APC)
        for g in range(GPW):
            b = g % 2
            rv = rows_v.at[b]
            pltpu.make_async_copy(x_hbm.at[idx_all.at[g]], rv, sems[b]).wait()
            for f in range(_FCH):
                acc = jnp.zeros((LANES,), jnp.float32)
                for e in range(APC):
                    acc = acc + rv[e, pl.ds(f * LANES, LANES)]
                out_v[g, pl.ds(f * LANES, LANES)] = acc * scale
            if g + 2 < GPW:
                fire(g + 2, b)
        pltpu.sync_copy(out_v, out_hbm.at[pl.ds(gbase, GPW)])

    return k(x, cidx_pad)


def _emb_body(af_ref, w_ref, b_ref, o_ref):
    o_ref[...] = (
        jnp.dot(af_ref[...], w_ref[...], preferred_element_type=jnp.float32)
        + b_ref[...]
    )


def _edge_w_body(nbrT_ref, w1t_ref, b1_ref, w2t_ref, o_ref):
    nbrT = nbrT_ref[...]
    outs = []
    for l in range(3):
        hT = jax.nn.softplus(
            jnp.dot(w1t_ref[l], nbrT, preferred_element_type=jnp.float32)
            + b1_ref[l]
        )
        sT = jnp.dot(w2t_ref[l], hT, preferred_element_type=jnp.float32)
        outs.append(sT)
    o_ref[...] = jnp.concatenate(outs, axis=0)


def _mm_body(x_ref, w_ref, o_ref):
    o_ref[...] = jnp.dot(
        x_ref[...], w_ref[...], preferred_element_type=jnp.float32
    )


def _bn_body(x_ref, g_ref, gm_ref, bt_ref, o_ref):
    pre = x_ref[...] + g_ref[...]
    mean = jnp.mean(pre, axis=0, keepdims=True)
    d = pre - mean
    var = jnp.mean(d * d, axis=0, keepdims=True)
    o_ref[...] = jax.nn.softplus(
        d / jnp.sqrt(var + 1e-5) * gm_ref[...] + bt_ref[...]
    )


def _readout_body(c_ref, wfc_ref, bfc_ref, wout_ref, bout_ref, o_ref, h_ref):
    h = jax.nn.softplus(
        jnp.dot(c_ref[...], wfc_ref[...], preferred_element_type=jnp.float32)
        + bfc_ref[...]
    )
    h_ref[...] = h
    o_ref[...] = (
        jnp.dot(h, wout_ref[...], preferred_element_type=jnp.float32)
        + bout_ref[...]
    )


def kernel(atom_fea, nbr_fea, nbr_idx, crystal_atom_idx, pos, params):
    del pos

    x = pl.pallas_call(
        _emb_body,
        out_shape=jax.ShapeDtypeStruct((N, AF), jnp.float32),
    )(atom_fea, params["W_emb"], params["b_emb"].reshape(1, AF))

    w1ts = jnp.stack([p["W1"].T for p in params["convs"]])
    b1s = jnp.stack([p["b1"].reshape(NF, 1) for p in params["convs"]])
    w2ts = jnp.stack([p["W2"][:, :1].T for p in params["convs"]])
    b2s = jnp.stack([p["b2"][0] for p in params["convs"]])

    E0 = N * M
    EB = 32000
    w_all = pl.pallas_call(
        _edge_w_body,
        grid=(E0 // EB,),
        in_specs=[
            pl.BlockSpec((NF, EB), lambda i: (0, i)),
            pl.BlockSpec((3, NF, NF), lambda i: (0, 0, 0)),
            pl.BlockSpec((3, NF, 1), lambda i: (0, 0, 0)),
            pl.BlockSpec((3, 1, NF), lambda i: (0, 0, 0)),
        ],
        out_specs=pl.BlockSpec((3, EB), lambda i: (0, i)),
        out_shape=jax.ShapeDtypeStruct((3, E0), jnp.float32),
    )(nbr_fea.reshape(E0, NF).T, w1ts, b1s, w2ts)
    w_all = (w_all + b2s.reshape(3, 1)) * EDGE_SCALE

    idx_flat = nbr_idx.astype(jnp.int32).reshape(E0)
    idx_pad = jnp.pad(idx_flat, (0, APAD * M - E0))
    w_pad = jnp.pad(w_all, ((0, 0), (0, APAD * M - E0)))

    for l, p in enumerate(params["convs"]):
        y = pl.pallas_call(
            _mm_body,
            out_shape=jax.ShapeDtypeStruct((N, AF), jnp.float32),
        )(x, p["Wtp"])
        g_pad = _conv_gather(y, idx_pad, w_pad[l])
        x = pl.pallas_call(
            _bn_body,
            out_shape=jax.ShapeDtypeStruct((N, AF), jnp.float32),
        )(
            x,
            g_pad[:N],
            p["gamma"].reshape(1, AF),
            p["beta"].reshape(1, AF),
        )

    cidx = crystal_atom_idx.astype(jnp.int32)
    cidx_pad = jnp.pad(cidx, ((0, GPAD - NCRY), (0, APC_PAD - APC)))
    crys = _crystal_pool(x, cidx_pad)[:NCRY]

    out, h = pl.pallas_call(
        _readout_body,
        out_shape=(
            jax.ShapeDtypeStruct((NCRY, 1), jnp.float32),
            jax.ShapeDtypeStruct((NCRY, H), jnp.float32),
        ),
    )(
        crys,
        params["W_fc"],
        params["b_fc"].reshape(1, H),
        params["W_out"],
        params["b_out"].reshape(1, 1),
    )
    return (out, h)
